# Initial kernel scaffold; baseline (speedup 1.0000x reference)
import jax
import jax.numpy as jnp
from jax.experimental import pallas as pl


def kernel(pos, edge_index, nbr_shift):
    raise NotImplementedError("write your pallas kernel here")


# speedup vs baseline: 1.9866x; 1.9866x over previous
import functools

import jax
import jax.numpy as jnp
from jax import lax
from jax.experimental import pallas as pl
from jax.experimental.pallas import tpu as pltpu
from jax.experimental.pallas import tpu_sc as plsc

N_NODES = 100000
N_EDGES = 3200000

NC = 2
NS = 16
NW = NC * NS
EPW = N_EDGES // NW
CHUNK = 2000
NCHUNK = EPW // CHUNK
NVEC = CHUNK // 16

SQRT3 = 3.0 ** 0.5
SQRT5 = 5.0 ** 0.5
SQRT15 = 15.0 ** 0.5


def _rsqrt(n2):
    i = plsc.bitcast(n2, jnp.int32)
    i = 0x5F3759DF - lax.shift_right_logical(i, 1)
    y = plsc.bitcast(i, jnp.float32)
    for _ in range(3):
        y = y * (1.5 - 0.5 * n2 * y * y)
    return y


def _make_sc_kernel():
    mesh = plsc.VectorSubcoreMesh(core_axis_name="c", subcore_axis_name="s")

    @functools.partial(
        pl.kernel,
        out_type=jax.ShapeDtypeStruct((N_EDGES * 9,), jnp.float32),
        mesh=mesh,
        compiler_params=pltpu.CompilerParams(needs_layout_passes=False),
        scratch_types=[
            pltpu.VMEM_SHARED((N_NODES,), jnp.float32),
            pltpu.VMEM_SHARED((N_NODES,), jnp.float32),
            pltpu.VMEM_SHARED((N_NODES,), jnp.float32),
            pltpu.VMEM((CHUNK,), jnp.int32),
            pltpu.VMEM((CHUNK,), jnp.int32),
            pltpu.VMEM((CHUNK,), jnp.float32),
            pltpu.VMEM((CHUNK,), jnp.float32),
            pltpu.VMEM((CHUNK,), jnp.float32),
            pltpu.VMEM((CHUNK,), jnp.float32),
            pltpu.VMEM((CHUNK,), jnp.float32),
            pltpu.VMEM((CHUNK,), jnp.float32),
            pltpu.VMEM((CHUNK * 3,), jnp.float32),
            pltpu.VMEM((CHUNK * 9,), jnp.float32),
            pltpu.VMEM((20000,), jnp.float32),
            pltpu.SemaphoreType.DMA,
        ],
    )
    def sh_kernel(pos_hbm, ei_hbm, shift_hbm, out_hbm,
                  posx_sp, posy_sp, posz_sp,
                  jv, iv, xj, yj, zj, xi, yi, zi, shift_v, out_v, bounce, sem):
        cid = lax.axis_index("c")
        sid = lax.axis_index("s")
        wid = sid * NC + cid
        base = wid * EPW

        PIECE = 20000
        NPIECE = N_NODES // PIECE

        @pl.when(sid == 0)
        def _stage_pos():
            for p, plane in enumerate((posx_sp, posy_sp, posz_sp)):
                def piece_body(t, _, p=p, plane=plane):
                    pltpu.sync_copy(
                        pos_hbm.at[pl.ds(p * N_NODES + t * PIECE, PIECE)], bounce)
                    pltpu.sync_copy(bounce, plane.at[pl.ds(t * PIECE, PIECE)])
                    return 0
                lax.fori_loop(0, NPIECE, piece_body, 0)

        plsc.subcore_barrier()

        iota16 = lax.broadcasted_iota(jnp.int32, (16,), 0)

        def chunk_body(k, _):
            b = base + k * CHUNK
            pltpu.sync_copy(ei_hbm.at[pl.ds(b, CHUNK)], jv)
            pltpu.sync_copy(ei_hbm.at[pl.ds(N_EDGES + b, CHUNK)], iv)
            cps = [
                pltpu.async_copy(posx_sp.at[jv], xj, sem),
                pltpu.async_copy(posy_sp.at[jv], yj, sem),
                pltpu.async_copy(posz_sp.at[jv], zj, sem),
                pltpu.async_copy(posx_sp.at[iv], xi, sem),
                pltpu.async_copy(posy_sp.at[iv], yi, sem),
                pltpu.async_copy(posz_sp.at[iv], zi, sem),
            ]
            pltpu.sync_copy(shift_hbm.at[pl.ds(b * 3, CHUNK * 3)], shift_v)
            for cp in cps:
                cp.wait()

            def vec_body(kk, _):
                e = kk * 16
                row = e + iota16
                r3 = row * 3
                vxj = xj[pl.ds(e, 16)]
                vyj = yj[pl.ds(e, 16)]
                vzj = zj[pl.ds(e, 16)]
                vxi = xi[pl.ds(e, 16)]
                vyi = yi[pl.ds(e, 16)]
                vzi = zi[pl.ds(e, 16)]
                sx = plsc.load_gather(shift_v, [r3])
                sy = plsc.load_gather(shift_v, [r3 + 1])
                sz = plsc.load_gather(shift_v, [r3 + 2])

                a0 = vxi + sx - vxj
                a1 = vyi + sy - vyj
                a2 = vzi + sz - vzj
                n2 = a0 * a0 + a1 * a1 + a2 * a2
                rn = _rsqrt(jnp.maximum(n2, 1e-24))
                x = a1 * rn
                y = a2 * rn
                z = a0 * rn

                x2 = x * x
                y2 = y * y
                z2 = z * z
                o1 = SQRT3 * x
                o2 = SQRT3 * y
                o3 = SQRT3 * z
                o4 = SQRT15 * x * z
                o5 = SQRT15 * x * y
                o6 = SQRT5 * (y2 - 0.5 * (x2 + z2))
                o7 = SQRT15 * y * z
                o8 = (SQRT15 / 2.0) * (z2 - x2)

                r9 = row * 9
                plsc.store_scatter(out_v, [r9], jnp.full((16,), 1.0, jnp.float32))
                plsc.store_scatter(out_v, [r9 + 1], o1)
                plsc.store_scatter(out_v, [r9 + 2], o2)
                plsc.store_scatter(out_v, [r9 + 3], o3)
                plsc.store_scatter(out_v, [r9 + 4], o4)
                plsc.store_scatter(out_v, [r9 + 5], o5)
                plsc.store_scatter(out_v, [r9 + 6], o6)
                plsc.store_scatter(out_v, [r9 + 7], o7)
                plsc.store_scatter(out_v, [r9 + 8], o8)
                return 0

            lax.fori_loop(0, NVEC, vec_body, 0)
            pltpu.sync_copy(out_v, out_hbm.at[pl.ds(b * 9, CHUNK * 9)])
            return 0

        lax.fori_loop(0, NCHUNK, chunk_body, 0)

    return sh_kernel


_sh_kernel = _make_sc_kernel()


@jax.jit
def kernel(pos, edge_index, nbr_shift):
    pos_planes = pos.T.reshape(3 * N_NODES)
    ei = edge_index.astype(jnp.int32).reshape(2 * N_EDGES)
    shift = nbr_shift.reshape(3 * N_EDGES)
    return _sh_kernel(pos_planes, ei, shift).reshape(N_EDGES, 9)
